# Initial kernel scaffold; baseline (speedup 1.0000x reference)
import jax
import jax.numpy as jnp
from jax.experimental import pallas as pl


def kernel(indices, other_features, table):
    raise NotImplementedError("write your pallas kernel here")


# speedup vs baseline: 2.5018x; 2.5018x over previous
import functools

import jax
import jax.numpy as jnp
from jax import lax
from jax.experimental import pallas as pl
from jax.experimental.pallas import tpu as pltpu
from jax.experimental.pallas import tpu_sc as plsc

D = 64
CH = 64
NBUF = 1
NC = 2
NS = 16
NW = NC * NS


def _sc_embed_concat(n_rows):
    per_w = n_rows // NW
    nch = per_w // CH
    mesh = plsc.VectorSubcoreMesh(core_axis_name="c", subcore_axis_name="s")

    @functools.partial(
        pl.kernel,
        out_type=jax.ShapeDtypeStruct((n_rows, 2, D), jnp.float32),
        mesh=mesh,
        compiler_params=pltpu.CompilerParams(use_tc_tiling_on_sc=False),
        scratch_types=[
            pltpu.VMEM((nch, CH), jnp.int32),
            pltpu.VMEM((NBUF, CH, D), jnp.float32),
            pltpu.VMEM((NBUF, CH, D), jnp.float32),
            pltpu.SemaphoreType.DMA,
            pltpu.SemaphoreType.DMA((NBUF,)),
            pltpu.SemaphoreType.DMA((NBUF,)),
            pltpu.SemaphoreType.DMA((NBUF,)),
            pltpu.SemaphoreType.DMA((NBUF,)),
        ],
    )
    def body(idx_hbm, other_hbm, table_hbm, out_hbm,
             idx_v, ebuf, obuf, sem_i, sem_g, sem_o, sem_we, sem_wo):
        wid = lax.axis_index("s") * NC + lax.axis_index("c")
        base = wid * per_w
        pltpu.async_copy(idx_hbm.at[wid], idx_v, sem_i).wait()

        def start_load(j, s):
            pltpu.make_async_copy(
                table_hbm.at[idx_v.at[j]], ebuf.at[s], sem_g.at[s]).start()
            pltpu.make_async_copy(
                other_hbm.at[pl.ds(base + j * CH, CH)], obuf.at[s],
                sem_o.at[s]).start()

        def wait_load(j, s):
            pltpu.make_async_copy(
                table_hbm.at[idx_v.at[j]], ebuf.at[s], sem_g.at[s]).wait()
            pltpu.make_async_copy(
                other_hbm.at[pl.ds(base + j * CH, CH)], obuf.at[s],
                sem_o.at[s]).wait()

        def start_write(j, s):
            r0 = base + j * CH
            pltpu.make_async_copy(
                ebuf.at[s], out_hbm.at[pl.ds(r0, CH), 0],
                sem_we.at[s]).start()
            pltpu.make_async_copy(
                obuf.at[s], out_hbm.at[pl.ds(r0, CH), 1],
                sem_wo.at[s]).start()

        def wait_write(s):
            pltpu.make_async_copy(
                ebuf.at[s], out_hbm.at[pl.ds(0, CH), 0],
                sem_we.at[s]).wait()
            pltpu.make_async_copy(
                obuf.at[s], out_hbm.at[pl.ds(0, CH), 1],
                sem_wo.at[s]).wait()

        def chunk(j, carry):
            start_load(j, 0)
            wait_load(j, 0)
            start_write(j, 0)
            wait_write(0)
            return carry

        lax.fori_loop(0, nch, chunk, 0)

    return body


def kernel(indices, other_features, table):
    b, l = indices.shape
    n_rows = b * l
    idx3 = indices.reshape(NW, n_rows // (NW * CH), CH).astype(jnp.int32)
    other2 = other_features.reshape(n_rows, D)
    out = _sc_embed_concat(n_rows)(idx3, other2, table)
    return out.reshape(b, l, 2 * D)


# speedup vs baseline: 2.8534x; 1.1405x over previous
import functools

import jax
import jax.numpy as jnp
from jax import lax
from jax.experimental import pallas as pl
from jax.experimental.pallas import tpu as pltpu
from jax.experimental.pallas import tpu_sc as plsc

D = 64
CH = 64
NBUF = 4
PF = 2
NC = 2
NS = 16
NW = NC * NS


def _sc_embed_concat(n_rows):
    per_w = n_rows // NW
    nch = per_w // CH
    mesh = plsc.VectorSubcoreMesh(core_axis_name="c", subcore_axis_name="s")

    @functools.partial(
        pl.kernel,
        out_type=jax.ShapeDtypeStruct((n_rows, 2, D), jnp.float32),
        mesh=mesh,
        compiler_params=pltpu.CompilerParams(use_tc_tiling_on_sc=False),
        scratch_types=[
            pltpu.VMEM((nch, CH), jnp.int32),
            pltpu.VMEM((NBUF, CH, D), jnp.float32),
            pltpu.VMEM((NBUF, CH, D), jnp.float32),
            pltpu.SemaphoreType.DMA,
            pltpu.SemaphoreType.DMA((NBUF,)),
            pltpu.SemaphoreType.DMA((NBUF,)),
            pltpu.SemaphoreType.DMA((NBUF,)),
            pltpu.SemaphoreType.DMA((NBUF,)),
        ],
    )
    def body(idx_hbm, other_hbm, table_hbm, out_hbm,
             idx_v, ebuf, obuf, sem_i, sem_g, sem_o, sem_we, sem_wo):
        wid = lax.axis_index("s") * NC + lax.axis_index("c")
        base = wid * per_w
        pltpu.async_copy(idx_hbm.at[wid], idx_v, sem_i).wait()

        def start_load(j, s):
            pltpu.make_async_copy(
                table_hbm.at[idx_v.at[j]], ebuf.at[s], sem_g.at[s]).start()
            pltpu.make_async_copy(
                other_hbm.at[pl.ds(base + j * CH, CH)], obuf.at[s],
                sem_o.at[s]).start()

        def wait_load(j, s):
            pltpu.make_async_copy(
                table_hbm.at[idx_v.at[j]], ebuf.at[s], sem_g.at[s]).wait()
            pltpu.make_async_copy(
                other_hbm.at[pl.ds(base + j * CH, CH)], obuf.at[s],
                sem_o.at[s]).wait()

        def start_write(j, s):
            r0 = base + j * CH
            pltpu.make_async_copy(
                ebuf.at[s], out_hbm.at[pl.ds(r0, CH), 0],
                sem_we.at[s]).start()
            pltpu.make_async_copy(
                obuf.at[s], out_hbm.at[pl.ds(r0, CH), 1],
                sem_wo.at[s]).start()

        def wait_write(s):
            pltpu.make_async_copy(
                ebuf.at[s], out_hbm.at[pl.ds(0, CH), 0],
                sem_we.at[s]).wait()
            pltpu.make_async_copy(
                obuf.at[s], out_hbm.at[pl.ds(0, CH), 1],
                sem_wo.at[s]).wait()

        for j in range(PF):
            start_load(j, j)
        for j in range(PF):
            wait_load(j, j)
            start_write(j, j)
            start_load(j + PF, (j + PF) % NBUF)

        n_steady = nch - 2 * PF
        assert n_steady % NBUF == 0

        def loop_body(k, carry):
            for b in range(NBUF):
                j = PF + k * NBUF + b
                s = (PF + b) % NBUF
                sp = (PF + b + PF) % NBUF
                wait_write(sp)
                start_load(j + PF, sp)
                wait_load(j, s)
                start_write(j, s)
            return carry

        lax.fori_loop(0, n_steady // NBUF, loop_body, 0)

        for j in range(nch - PF, nch):
            wait_load(j, j % NBUF)
            start_write(j, j % NBUF)
        for s in range(NBUF):
            wait_write(s)

    return body


def kernel(indices, other_features, table):
    b, l = indices.shape
    n_rows = b * l
    idx3 = indices.reshape(NW, n_rows // (NW * CH), CH).astype(jnp.int32)
    other2 = other_features.reshape(n_rows, D)
    out = _sc_embed_concat(n_rows)(idx3, other2, table)
    return out.reshape(b, l, 2 * D)


# speedup vs baseline: 3.9934x; 1.3995x over previous
import functools

import jax
import jax.numpy as jnp
from jax import lax
from jax.experimental import pallas as pl
from jax.experimental.pallas import tpu as pltpu
from jax.experimental.pallas import tpu_sc as plsc

D = 64
NBUF = 8
PF = 4
NC = 2
NS = 16
NW = NC * NS
RU = 5


def _sc_embed_concat(b, l):
    per_w = b // NW
    mesh = plsc.VectorSubcoreMesh(core_axis_name="c", subcore_axis_name="s")

    @functools.partial(
        pl.kernel,
        out_type=jax.ShapeDtypeStruct((b, l, 2 * D), jnp.float32),
        mesh=mesh,
        scratch_types=[
            pltpu.VMEM((per_w, l), jnp.int32),
            pltpu.VMEM((NBUF, l, 2 * D), jnp.float32),
            pltpu.VMEM((NBUF, l, D), jnp.float32),
            pltpu.SemaphoreType.DMA,
            pltpu.SemaphoreType.DMA((NBUF,)),
            pltpu.SemaphoreType.DMA((NBUF,)),
            pltpu.SemaphoreType.DMA((NBUF,)),
        ],
    )
    def body(idx_hbm, other_hbm, table_hbm, out_hbm,
             idx_v, cbuf, obuf, sem_i, sem_g, sem_o, sem_w):
        wid = lax.axis_index("s") * NC + lax.axis_index("c")
        base = wid * per_w
        pltpu.async_copy(idx_hbm.at[pl.ds(base, per_w)], idx_v, sem_i).wait()

        def start_load(j, s):
            pltpu.make_async_copy(
                table_hbm.at[idx_v.at[j]], cbuf.at[s], sem_g.at[s]).start()
            pltpu.make_async_copy(
                other_hbm.at[base + j], obuf.at[s], sem_o.at[s]).start()

        def wait_load(j, s):
            pltpu.make_async_copy(
                table_hbm.at[idx_v.at[j]], cbuf.at[s], sem_g.at[s]).wait()
            pltpu.make_async_copy(
                other_hbm.at[base + j], obuf.at[s], sem_o.at[s]).wait()

        def start_write(j, s):
            def rows(i, carry):
                for k in range(RU):
                    r = i * RU + k
                    for c in range(D // 16):
                        cbuf[s, r, pl.ds(D + c * 16, 16)] = (
                            obuf[s, r, pl.ds(c * 16, 16)])
                return carry

            lax.fori_loop(0, l // RU, rows, 0)
            pltpu.make_async_copy(
                cbuf.at[s], out_hbm.at[base + j], sem_w.at[s]).start()

        def wait_write(s):
            pltpu.make_async_copy(
                cbuf.at[s], out_hbm.at[0], sem_w.at[s]).wait()

        for j in range(PF):
            start_load(j, j)
        for j in range(PF):
            wait_load(j, j)
            start_write(j, j)
            start_load(j + PF, (j + PF) % NBUF)

        n_steady = per_w - 2 * PF
        assert n_steady % NBUF == 0

        def loop_body(k, carry):
            for bb in range(NBUF):
                j = PF + k * NBUF + bb
                s = (PF + bb) % NBUF
                sp = (PF + bb + PF) % NBUF
                wait_write(sp)
                start_load(j + PF, sp)
                wait_load(j, s)
                start_write(j, s)
            return carry

        lax.fori_loop(0, n_steady // NBUF, loop_body, 0)

        for j in range(per_w - PF, per_w):
            wait_load(j, j % NBUF)
            start_write(j, j % NBUF)
        for s in range(NBUF):
            wait_write(s)

    return body


def kernel(indices, other_features, table):
    b, l = indices.shape
    table_pad = jnp.pad(table, ((0, 0), (0, D)))
    return _sc_embed_concat(b, l)(
        indices.astype(jnp.int32), other_features, table_pad)


# speedup vs baseline: 5.9286x; 1.4846x over previous
import functools

import jax
import jax.numpy as jnp
from jax import lax
from jax.experimental import pallas as pl
from jax.experimental.pallas import tpu as pltpu
from jax.experimental.pallas import tpu_sc as plsc

D = 64
CH = 64
NBUF = 6
PF = 3
LAG = NBUF - PF
NC = 2
NS = 16
NW = NC * NS
RU = 4


def _sc_embed_concat(b, l):
    slab = b // NW
    nch = l * (slab // CH)
    mesh = plsc.VectorSubcoreMesh(core_axis_name="c", subcore_axis_name="s")

    @functools.partial(
        pl.kernel,
        out_type=jax.ShapeDtypeStruct((l, b, 2 * D), jnp.float32),
        mesh=mesh,
        scratch_types=[
            pltpu.VMEM((l, slab), jnp.int32),
            pltpu.VMEM((NBUF, CH, 2 * D), jnp.float32),
            pltpu.VMEM((NBUF, CH, D), jnp.float32),
            pltpu.SemaphoreType.DMA,
            pltpu.SemaphoreType.DMA((NBUF,)),
            pltpu.SemaphoreType.DMA((NBUF,)),
            pltpu.SemaphoreType.DMA((NBUF,)),
        ],
    )
    def body(idx_hbm, other_hbm, table_hbm, out_hbm,
             idx_v, cbuf, obuf, sem_i, sem_g, sem_o, sem_w):
        wid = lax.axis_index("s") * NC + lax.axis_index("c")
        base = pl.multiple_of(wid * slab, slab)
        pltpu.async_copy(
            idx_hbm.at[:, pl.ds(base, slab)], idx_v, sem_i).wait()

        nh = slab // CH

        def addr(j):
            return j // nh, (j % nh) * CH

        def start_load(j, s):
            li, off = addr(j)
            b0 = pl.multiple_of(base + off, CH)
            pltpu.make_async_copy(
                table_hbm.at[idx_v.at[li, pl.ds(off, CH)]], cbuf.at[s],
                sem_g.at[s]).start()
            pltpu.make_async_copy(
                other_hbm.at[li, pl.ds(b0, CH)], obuf.at[s],
                sem_o.at[s]).start()

        def wait_load(j, s):
            li, off = addr(j)
            b0 = pl.multiple_of(base + off, CH)
            pltpu.make_async_copy(
                table_hbm.at[idx_v.at[li, pl.ds(off, CH)]], cbuf.at[s],
                sem_g.at[s]).wait()
            pltpu.make_async_copy(
                other_hbm.at[li, pl.ds(b0, CH)], obuf.at[s],
                sem_o.at[s]).wait()

        def start_write(j, s):
            def rows(i, carry):
                for k in range(RU):
                    r = i * RU + k
                    for c in range(D // 16):
                        cbuf[s, r, pl.ds(D + c * 16, 16)] = (
                            obuf[s, r, pl.ds(c * 16, 16)])
                return carry

            lax.fori_loop(0, CH // RU, rows, 0)
            li, off = addr(j)
            b0 = pl.multiple_of(base + off, CH)
            pltpu.make_async_copy(
                cbuf.at[s], out_hbm.at[li, pl.ds(b0, CH)],
                sem_w.at[s]).start()

        def wait_write(s):
            pltpu.make_async_copy(
                cbuf.at[s], out_hbm.at[0, pl.ds(0, CH)], sem_w.at[s]).wait()

        def step(j, s, drain, prefetch):
            if drain:
                wait_write((s - LAG) % NBUF)
            if prefetch:
                start_load(j + PF, (s + PF) % NBUF)
            wait_load(j, s)
            start_write(j, s)

        for j in range(PF):
            start_load(j, j)

        w0 = NBUF
        w1 = w0 + ((nch - PF - w0) // NBUF) * NBUF
        for j in range(w0):
            step(j, j % NBUF, drain=j >= LAG, prefetch=j + PF < nch)

        def loop_body(k, carry):
            for bb in range(NBUF):
                step(w0 + k * NBUF + bb, bb, drain=True, prefetch=True)
            return carry

        lax.fori_loop(0, (w1 - w0) // NBUF, loop_body, 0)

        for j in range(w1, nch):
            step(j, j % NBUF, drain=j >= LAG, prefetch=j + PF < nch)
        for j in range(nch - LAG, nch):
            wait_write(j % NBUF)

    return body


def kernel(indices, other_features, table):
    b, l = indices.shape
    idx_t = jnp.transpose(indices).astype(jnp.int32)
    other_t = jnp.transpose(other_features, (1, 0, 2))
    table_pad = jnp.concatenate(
        [table, jnp.zeros_like(table)], axis=1)
    out_t = _sc_embed_concat(b, l)(idx_t, other_t, table_pad)
    return jnp.transpose(out_t, (1, 0, 2))
